# baseline (speedup 1.0000x reference)
import jax
import jax.numpy as jnp
from jax import lax
from jax.experimental import pallas as pl
from jax.experimental.pallas import tpu as pltpu

_NE = 1024
_BETA = 0.25


_CHUNKS = 4


def _vq_body(z_ref, emb_ref, zq_ref, idx_ref, loss_ref, acc_ref):
    i = pl.program_id(0)
    z = z_ref[0]
    emb = emb_ref[...]
    T = z.shape[0]
    C = T // _CHUNKS
    e_sq = jnp.sum(emb**2, axis=1)
    eids = lax.broadcasted_iota(jnp.int32, (C, _NE), 1).astype(jnp.float32)
    onehots, parts = [], []
    for h in range(_CHUNKS):
        zh = z[h * C:(h + 1) * C]
        s2 = lax.dot_general(
            zh * -2.0, emb, (((1,), (1,)), ((), ())),
            preferred_element_type=jnp.float32,
        )
        z_sq = jnp.sum(zh**2, axis=1, keepdims=True)
        dist = (z_sq + s2) + e_sq[None, :]
        dmin = jnp.min(dist, axis=1, keepdims=True)
        idxf = jnp.min(jnp.where(dist == dmin, eids, float(_NE)), axis=1)
        onehots.append(jnp.where(eids == idxf[:, None], 1.0, 0.0))
        parts.append(jnp.sum(dmin))
    onehot = jnp.concatenate(onehots, axis=0)
    z_q = lax.dot_general(
        onehot, emb, (((1,), (0,)), ((), ())),
        preferred_element_type=jnp.float32,
    )
    zq_ref[0] = z_q
    iota = lax.broadcasted_iota(jnp.int32, (2, _NE), 1)
    hilo = jnp.where(
        lax.broadcasted_iota(jnp.int32, (2, _NE), 0) == 0,
        iota // 256, iota % 256,
    ).astype(jnp.float32)
    hilo_row = lax.dot_general(
        hilo, onehot, (((1,), (1,)), ((), ())),
        preferred_element_type=jnp.float32,
    )
    idx_row = hilo_row[0:1, :] * 256.0 + hilo_row[1:2, :]
    idx_ref[pl.ds(i, 1), :] = idx_row.astype(jnp.int32)
    part = sum(parts)

    @pl.when(i == 0)
    def _init():
        acc_ref[0] = part

    @pl.when(i > 0)
    def _accum():
        acc_ref[0] += part

    @pl.when(i == pl.num_programs(0) - 1)
    def _fin():
        n_elems = pl.num_programs(0) * z.shape[0] * z.shape[1]
        loss_ref[0, 0] = acc_ref[0] * ((1.0 + _BETA) / n_elems)


def kernel(z, emb_weight):
    B, T, D = z.shape
    z_q, idx2, loss2 = pl.pallas_call(
        _vq_body,
        grid=(B,),
        in_specs=[
            pl.BlockSpec((1, T, D), lambda i: (i, 0, 0)),
            pl.BlockSpec((_NE, D), lambda i: (0, 0)),
        ],
        out_specs=[
            pl.BlockSpec((1, T, D), lambda i: (i, 0, 0)),
            pl.BlockSpec((B, T), lambda i: (0, 0)),
            pl.BlockSpec(memory_space=pltpu.SMEM),
        ],
        out_shape=[
            jax.ShapeDtypeStruct((B, T, D), jnp.float32),
            jax.ShapeDtypeStruct((B, T), jnp.int32),
            jax.ShapeDtypeStruct((1, 1), jnp.float32),
        ],
        scratch_shapes=[pltpu.SMEM((1,), jnp.float32)],
    )(z, emb_weight)
    return z_q, loss2[0, 0], idx2


# speedup vs baseline: 1.0783x; 1.0783x over previous
import jax
import jax.numpy as jnp
from jax import lax
from jax.experimental import pallas as pl
from jax.experimental.pallas import tpu as pltpu

_NE = 1024
_BETA = 0.25
_ROWS = 2
_CHUNKS = 4


def _vq_body(z_ref, emb_ref, zq_ref, idx_ref, loss_ref, acc_ref):
    i = pl.program_id(0)
    zb = z_ref[...]
    R, T, D = zb.shape
    z = zb.reshape(R * T, D)
    emb = emb_ref[...]
    N = R * T
    C = N // _CHUNKS
    e_sq = jnp.sum(emb**2, axis=1)
    s2 = lax.dot_general(
        z * -2.0, emb, (((1,), (1,)), ((), ())),
        preferred_element_type=jnp.float32,
    )
    eids = lax.broadcasted_iota(jnp.int32, (C, _NE), 1).astype(jnp.float32)
    onehots, parts = [], []
    for h in range(_CHUNKS):
        zh = z[h * C:(h + 1) * C]
        z_sq = jnp.sum(zh**2, axis=1, keepdims=True)
        dist = (z_sq + s2[h * C:(h + 1) * C]) + e_sq[None, :]
        dmin = jnp.min(dist, axis=1, keepdims=True)
        idxf = jnp.min(jnp.where(dist == dmin, eids, float(_NE)), axis=1)
        onehots.append(jnp.where(eids == idxf[:, None], 1.0, 0.0))
        parts.append(jnp.sum(dmin))
    onehot = jnp.concatenate(onehots, axis=0)
    z_q = lax.dot_general(
        onehot, emb, (((1,), (0,)), ((), ())),
        preferred_element_type=jnp.float32,
    )
    zq_ref[...] = z_q.reshape(R, T, D)
    iota = lax.broadcasted_iota(jnp.int32, (2, _NE), 1)
    hilo = jnp.where(
        lax.broadcasted_iota(jnp.int32, (2, _NE), 0) == 0,
        iota // 256, iota % 256,
    ).astype(jnp.float32)
    idx_rows = []
    for r in range(R):
        hilo_row = lax.dot_general(
            hilo, onehot[r * T:(r + 1) * T], (((1,), (1,)), ((), ())),
            preferred_element_type=jnp.float32,
        )
        idx_row = hilo_row[0:1, :] * 256.0 + hilo_row[1:2, :]
        idx_ref[pl.ds(i * R + r, 1), :] = idx_row.astype(jnp.int32)
    del idx_rows
    part = sum(parts)

    @pl.when(i == 0)
    def _init():
        acc_ref[0] = part

    @pl.when(i > 0)
    def _accum():
        acc_ref[0] += part

    @pl.when(i == pl.num_programs(0) - 1)
    def _fin():
        n_elems = pl.num_programs(0) * N * D
        loss_ref[0, 0] = acc_ref[0] * ((1.0 + _BETA) / n_elems)


def kernel(z, emb_weight):
    B, T, D = z.shape
    z_q, idx2, loss2 = pl.pallas_call(
        _vq_body,
        grid=(B // _ROWS,),
        in_specs=[
            pl.BlockSpec((_ROWS, T, D), lambda i: (i, 0, 0)),
            pl.BlockSpec((_NE, D), lambda i: (0, 0)),
        ],
        out_specs=[
            pl.BlockSpec((_ROWS, T, D), lambda i: (i, 0, 0)),
            pl.BlockSpec((B, T), lambda i: (0, 0)),
            pl.BlockSpec(memory_space=pltpu.SMEM),
        ],
        out_shape=[
            jax.ShapeDtypeStruct((B, T, D), jnp.float32),
            jax.ShapeDtypeStruct((B, T), jnp.int32),
            jax.ShapeDtypeStruct((1, 1), jnp.float32),
        ],
        scratch_shapes=[pltpu.SMEM((1,), jnp.float32)],
    )(z, emb_weight)
    return z_q, loss2[0, 0], idx2


# speedup vs baseline: 1.0921x; 1.0129x over previous
import jax
import jax.numpy as jnp
from jax import lax
from jax.experimental import pallas as pl
from jax.experimental.pallas import tpu as pltpu

_NE = 1024
_BETA = 0.25
_ROWS = 4
_CHUNKS = 4


def _vq_body(z_ref, emb_ref, zq_ref, idx_ref, loss_ref, acc_ref):
    i = pl.program_id(0)
    zb = z_ref[...]
    R, T, D = zb.shape
    z = zb.reshape(R * T, D)
    emb = emb_ref[...]
    N = R * T
    C = N // _CHUNKS
    e_sq = jnp.sum(emb**2, axis=1)
    s2 = lax.dot_general(
        z * -2.0, emb, (((1,), (1,)), ((), ())),
        preferred_element_type=jnp.float32,
    )
    eids = lax.broadcasted_iota(jnp.int32, (C, _NE), 1).astype(jnp.float32)
    onehots, parts = [], []
    for h in range(_CHUNKS):
        zh = z[h * C:(h + 1) * C]
        z_sq = jnp.sum(zh**2, axis=1, keepdims=True)
        dist = (z_sq + s2[h * C:(h + 1) * C]) + e_sq[None, :]
        dmin = jnp.min(dist, axis=1, keepdims=True)
        idxf = jnp.min(jnp.where(dist == dmin, eids, float(_NE)), axis=1)
        onehots.append(jnp.where(eids == idxf[:, None], 1.0, 0.0))
        parts.append(jnp.sum(dmin))
    onehot = jnp.concatenate(onehots, axis=0)
    z_q = lax.dot_general(
        onehot, emb, (((1,), (0,)), ((), ())),
        preferred_element_type=jnp.float32,
    )
    zq_ref[...] = z_q.reshape(R, T, D)
    iota = lax.broadcasted_iota(jnp.int32, (2, _NE), 1)
    hilo = jnp.where(
        lax.broadcasted_iota(jnp.int32, (2, _NE), 0) == 0,
        iota // 256, iota % 256,
    ).astype(jnp.float32)
    idx_rows = []
    for r in range(R):
        hilo_row = lax.dot_general(
            hilo, onehot[r * T:(r + 1) * T], (((1,), (1,)), ((), ())),
            preferred_element_type=jnp.float32,
        )
        idx_row = hilo_row[0:1, :] * 256.0 + hilo_row[1:2, :]
        idx_ref[pl.ds(i * R + r, 1), :] = idx_row.astype(jnp.int32)
    del idx_rows
    part = sum(parts)

    @pl.when(i == 0)
    def _init():
        acc_ref[0] = part

    @pl.when(i > 0)
    def _accum():
        acc_ref[0] += part

    @pl.when(i == pl.num_programs(0) - 1)
    def _fin():
        n_elems = pl.num_programs(0) * N * D
        loss_ref[0, 0] = acc_ref[0] * ((1.0 + _BETA) / n_elems)


def kernel(z, emb_weight):
    B, T, D = z.shape
    z_q, idx2, loss2 = pl.pallas_call(
        _vq_body,
        grid=(B // _ROWS,),
        in_specs=[
            pl.BlockSpec((_ROWS, T, D), lambda i: (i, 0, 0)),
            pl.BlockSpec((_NE, D), lambda i: (0, 0)),
        ],
        out_specs=[
            pl.BlockSpec((_ROWS, T, D), lambda i: (i, 0, 0)),
            pl.BlockSpec((B, T), lambda i: (0, 0)),
            pl.BlockSpec(memory_space=pltpu.SMEM),
        ],
        out_shape=[
            jax.ShapeDtypeStruct((B, T, D), jnp.float32),
            jax.ShapeDtypeStruct((B, T), jnp.int32),
            jax.ShapeDtypeStruct((1, 1), jnp.float32),
        ],
        scratch_shapes=[pltpu.SMEM((1,), jnp.float32)],
    )(z, emb_weight)
    return z_q, loss2[0, 0], idx2


# speedup vs baseline: 1.1203x; 1.0258x over previous
import jax
import jax.numpy as jnp
from jax import lax
from jax.experimental import pallas as pl
from jax.experimental.pallas import tpu as pltpu

_NE = 1024
_BETA = 0.25
_ROWS = 4
_CHUNKS = 3


def _vq_body(z_ref, emb_ref, zq_ref, idx_ref, loss_ref, acc_ref):
    i = pl.program_id(0)
    zb = z_ref[...]
    R, T, D = zb.shape
    z = zb.reshape(R * T, D)
    emb = emb_ref[...]
    N = R * T
    C = N // _CHUNKS
    e_sq = jnp.sum(emb**2, axis=1)
    s2 = lax.dot_general(
        z * -2.0, emb, (((1,), (1,)), ((), ())),
        preferred_element_type=jnp.float32,
    )
    eids = lax.broadcasted_iota(jnp.int32, (C, _NE), 1).astype(jnp.float32)
    onehots, parts = [], []
    for h in range(_CHUNKS):
        zh = z[h * C:(h + 1) * C]
        z_sq = jnp.sum(zh**2, axis=1, keepdims=True)
        dist = (z_sq + s2[h * C:(h + 1) * C]) + e_sq[None, :]
        dmin = jnp.min(dist, axis=1, keepdims=True)
        idxf = jnp.min(jnp.where(dist == dmin, eids, float(_NE)), axis=1)
        oh = jnp.where(eids == idxf[:, None], 1.0, 0.0)
        onehots.append(oh)
        parts.append(jnp.sum(dmin))
    onehot = jnp.concatenate(onehots, axis=0)
    zq_chunks = [
        lax.dot_general(
            oh, emb, (((1,), (0,)), ((), ())),
            preferred_element_type=jnp.float32,
        )
        for oh in onehots
    ]
    z_q = jnp.concatenate(zq_chunks, axis=0)
    zq_ref[...] = z_q.reshape(R, T, D)
    iota = lax.broadcasted_iota(jnp.int32, (2, _NE), 1)
    hilo = jnp.where(
        lax.broadcasted_iota(jnp.int32, (2, _NE), 0) == 0,
        iota // 256, iota % 256,
    ).astype(jnp.float32)
    idx_rows = []
    for r in range(R):
        hilo_row = lax.dot_general(
            hilo, onehot[r * T:(r + 1) * T], (((1,), (1,)), ((), ())),
            preferred_element_type=jnp.float32,
        )
        idx_row = hilo_row[0:1, :] * 256.0 + hilo_row[1:2, :]
        idx_ref[pl.ds(i * R + r, 1), :] = idx_row.astype(jnp.int32)
    del idx_rows
    part = sum(parts)

    @pl.when(i == 0)
    def _init():
        acc_ref[0] = part

    @pl.when(i > 0)
    def _accum():
        acc_ref[0] += part

    @pl.when(i == pl.num_programs(0) - 1)
    def _fin():
        n_elems = pl.num_programs(0) * N * D
        loss_ref[0, 0] = acc_ref[0] * ((1.0 + _BETA) / n_elems)


def kernel(z, emb_weight):
    B, T, D = z.shape
    z_q, idx2, loss2 = pl.pallas_call(
        _vq_body,
        grid=(B // _ROWS,),
        in_specs=[
            pl.BlockSpec((_ROWS, T, D), lambda i: (i, 0, 0)),
            pl.BlockSpec((_NE, D), lambda i: (0, 0)),
        ],
        out_specs=[
            pl.BlockSpec((_ROWS, T, D), lambda i: (i, 0, 0)),
            pl.BlockSpec((B, T), lambda i: (0, 0)),
            pl.BlockSpec(memory_space=pltpu.SMEM),
        ],
        out_shape=[
            jax.ShapeDtypeStruct((B, T, D), jnp.float32),
            jax.ShapeDtypeStruct((B, T), jnp.int32),
            jax.ShapeDtypeStruct((1, 1), jnp.float32),
        ],
        scratch_shapes=[pltpu.SMEM((1,), jnp.float32)],
    )(z, emb_weight)
    return z_q, loss2[0, 0], idx2
